# baseline (speedup 1.0000x reference)
import functools

import jax
import jax.numpy as jnp
from jax import lax
from jax.experimental import pallas as pl
from jax.experimental.pallas import tpu as pltpu
from jax.experimental.pallas import tpu_sc as plsc

_LANES = 16
_K = 128


def _sc_gather(total, C, N, S, NW):
    per_w = total // NW
    J = per_w // _K
    mesh = plsc.VectorSubcoreMesh(core_axis_name="c", subcore_axis_name="s")
    NC = 2

    @functools.partial(
        pl.kernel,
        mesh=mesh,
        out_type=jax.ShapeDtypeStruct((C, total // _K, _K), jnp.float32),
        scratch_types=[
            pltpu.VMEM((J, _K), jnp.int32),
            pltpu.VMEM((C, J, _K), jnp.int32),
            pltpu.VMEM((C, J, _K), jnp.float32),
            pltpu.SemaphoreType.DMA,
        ],
        compiler_params=pltpu.CompilerParams(use_tc_tiling_on_sc=False),
    )
    def body(pts_hbm, idx_hbm, out_hbm, idx_v, widx_v, vals_v, sem):
        wid = lax.axis_index("s") * NC + lax.axis_index("c")
        row0 = wid * J
        pltpu.sync_copy(idx_hbm.at[pl.ds(row0, J)], idx_v)

        woff = (wid * per_w) // S * (C * N)
        off = jnp.full((_LANES,), woff, dtype=jnp.int32)
        three = jnp.full((_LANES,), C, dtype=jnp.int32)
        for j in range(J):
            for k in range(_K // _LANES):
                sl = pl.ds(k * _LANES, _LANES)
                base = idx_v[j, sl] * three + off
                for c in range(C):
                    widx_v[c, j, sl] = base + c

        copies = [
            pltpu.async_copy(pts_hbm.at[widx_v.at[c, j]], vals_v.at[c, j], sem)
            for c in range(C)
            for j in range(J)
        ]
        for cp in copies:
            cp.wait()

        for c in range(C):
            pltpu.sync_copy(vals_v.at[c], out_hbm.at[c].at[pl.ds(row0, J)])

    return body


def kernel(points, fps_idx):
    B, N, C = points.shape
    S = fps_idx.shape[1]
    total = B * S
    NW = 32

    pts_flat = points.reshape(B * N * C)
    idx2d = fps_idx.reshape(total // _K, _K).astype(jnp.int32)

    out = _sc_gather(total, C, N, S, NW)(pts_flat, idx2d)
    return jnp.transpose(out.reshape(C, B, S), (1, 2, 0))


# speedup vs baseline: 1.0048x; 1.0048x over previous
import functools

import jax
import jax.numpy as jnp
from jax import lax
from jax.experimental import pallas as pl
from jax.experimental.pallas import tpu as pltpu
from jax.experimental.pallas import tpu_sc as plsc

_LANES = 16
_K = 128


def _sc_gather(total, C, N, S, NW):
    per_w = total // NW
    J = per_w // _K
    mesh = plsc.VectorSubcoreMesh(core_axis_name="c", subcore_axis_name="s")
    NC = 2

    @functools.partial(
        pl.kernel,
        mesh=mesh,
        out_type=jax.ShapeDtypeStruct((C, total // _K, _K), jnp.float32),
        scratch_types=[
            pltpu.VMEM((J, _K), jnp.int32),
            pltpu.VMEM((C, J, _K), jnp.int32),
            pltpu.VMEM((C, J, _K), jnp.float32),
            pltpu.SemaphoreType.DMA,
        ],
    )
    def body(pts_hbm, idx_hbm, out_hbm, idx_v, widx_v, vals_v, sem):
        wid = lax.axis_index("s") * NC + lax.axis_index("c")
        row0 = wid * J
        pltpu.sync_copy(idx_hbm.at[pl.ds(row0, J)], idx_v)

        woff = (wid * per_w) // S * (C * N)
        off = jnp.full((_LANES,), woff, dtype=jnp.int32)
        three = jnp.full((_LANES,), C, dtype=jnp.int32)
        for j in range(J):
            for k in range(_K // _LANES):
                sl = pl.ds(k * _LANES, _LANES)
                base = idx_v[j, sl] * three + off
                for c in range(C):
                    widx_v[c, j, sl] = base + c

        copies = [
            pltpu.async_copy(pts_hbm.at[widx_v.at[c, j]], vals_v.at[c, j], sem)
            for c in range(C)
            for j in range(J)
        ]
        for cp in copies:
            cp.wait()

        for c in range(C):
            pltpu.sync_copy(vals_v.at[c], out_hbm.at[c].at[pl.ds(row0, J)])

    return body


def kernel(points, fps_idx):
    B, N, C = points.shape
    S = fps_idx.shape[1]
    total = B * S
    NW = 32

    pts_flat = points.reshape(B * N * C)
    idx2d = fps_idx.reshape(total // _K, _K).astype(jnp.int32)

    out = _sc_gather(total, C, N, S, NW)(pts_flat, idx2d)
    return jnp.transpose(out.reshape(C, B, S), (1, 2, 0))


# speedup vs baseline: 90.9643x; 90.5342x over previous
import functools

import jax
import jax.numpy as jnp
from jax import lax
from jax.experimental import pallas as pl
from jax.experimental.pallas import tpu as pltpu
from jax.experimental.pallas import tpu_sc as plsc

_LANES = 16
_K = 128


def _sc_gather(total, C, N, S, NW):
    per_w = total // NW
    J = per_w // _K
    mesh = plsc.VectorSubcoreMesh(core_axis_name="c", subcore_axis_name="s")
    NC = 2

    @functools.partial(
        pl.kernel,
        mesh=mesh,
        out_type=jax.ShapeDtypeStruct((C, total // _K, _K), jnp.float32),
        scratch_types=[
            pltpu.VMEM((J, _K), jnp.int32),
            pltpu.VMEM((C, J, _K), jnp.int32),
            pltpu.VMEM((C, J, _K), jnp.float32),
            pltpu.SemaphoreType.DMA,
        ],
    )
    def body(pts_hbm, idx_hbm, out_hbm, idx_v, widx_v, vals_v, sem):
        wid = lax.axis_index("s") * NC + lax.axis_index("c")
        row0 = wid * J
        pltpu.sync_copy(idx_hbm.at[pl.ds(row0, J)], idx_v)

        woff = (wid * per_w) // S * N
        off = jnp.full((_LANES,), woff, dtype=jnp.int32)
        plane = N * (total // S)
        for j in range(J):
            for k in range(_K // _LANES):
                sl = pl.ds(k * _LANES, _LANES)
                base = idx_v[j, sl] + off
                for c in range(C):
                    widx_v[c, j, sl] = base + c * plane

        copies = [
            pltpu.async_copy(pts_hbm.at[widx_v.at[c, j]], vals_v.at[c, j], sem)
            for c in range(C)
            for j in range(J)
        ]
        for cp in copies:
            cp.wait()

        for c in range(C):
            pltpu.sync_copy(vals_v.at[c], out_hbm.at[c].at[pl.ds(row0, J)])

    return body


def kernel(points, fps_idx):
    B, N, C = points.shape
    S = fps_idx.shape[1]
    total = B * S
    NW = 32

    pts_flat = jnp.transpose(points, (2, 0, 1)).reshape(C * B * N)
    idx2d = fps_idx.reshape(total // _K, _K).astype(jnp.int32)

    out = _sc_gather(total, C, N, S, NW)(pts_flat, idx2d)
    return jnp.transpose(out.reshape(C, B, S), (1, 2, 0))


# speedup vs baseline: 137.7885x; 1.5148x over previous
import functools

import jax
import jax.numpy as jnp
from jax import lax
from jax.experimental import pallas as pl
from jax.experimental.pallas import tpu as pltpu
from jax.experimental.pallas import tpu_sc as plsc

_LANES = 16
_K = 128


def _sc_gather(B, N, C, S):
    total = B * S
    R = S // _K
    mesh = plsc.VectorSubcoreMesh(core_axis_name="c", subcore_axis_name="s")
    NC = 2
    NW = 32

    @functools.partial(
        pl.kernel,
        mesh=mesh,
        out_type=jax.ShapeDtypeStruct((C, total // _K, _K), jnp.float32),
        scratch_types=[
            pltpu.VMEM((N,), jnp.float32),
            pltpu.VMEM((R, _K), jnp.int32),
            pltpu.VMEM((R, _K), jnp.float32),
        ],
        compiler_params=pltpu.CompilerParams(needs_layout_passes=False),
    )
    def body(pts_hbm, idx_hbm, out_hbm, plane_v, idx_v, vals_v):
        wid = lax.axis_index("s") * NC + lax.axis_index("c")

        def task(c, b):
            pltpu.sync_copy(pts_hbm.at[c, b], plane_v)
            pltpu.sync_copy(idx_hbm.at[pl.ds(b * R, R)], idx_v)
            for r in range(R):
                for k in range(_K // _LANES):
                    sl = pl.ds(k * _LANES, _LANES)
                    ids = idx_v[r, sl]
                    vals_v[r, sl] = plsc.load_gather(plane_v, [ids])
            pltpu.sync_copy(
                vals_v, out_hbm.at[c].at[pl.ds(b * R, R)])

        task(wid // B, wid % B)

        @pl.when(wid < C * B - NW)
        def _():
            t = wid + NW
            task(t // B, t % B)

    return body


def kernel(points, fps_idx):
    B, N, C = points.shape
    S = fps_idx.shape[1]

    pts3 = jnp.transpose(points, (2, 0, 1))
    idx2d = fps_idx.reshape(B * S // _K, _K).astype(jnp.int32)

    out = _sc_gather(B, N, C, S)(pts3, idx2d)
    return jnp.transpose(out.reshape(C, B, S), (1, 2, 0))


# speedup vs baseline: 142.5379x; 1.0345x over previous
import functools

import jax
import jax.numpy as jnp
from jax import lax
from jax.experimental import pallas as pl
from jax.experimental.pallas import tpu as pltpu
from jax.experimental.pallas import tpu_sc as plsc

_LANES = 16
_K = 128


def _sc_gather(B, N, C, S):
    total = B * S
    R = S // _K
    mesh = plsc.VectorSubcoreMesh(core_axis_name="c", subcore_axis_name="s")
    NC = 2
    NW = 32

    @functools.partial(
        pl.kernel,
        mesh=mesh,
        out_type=jax.ShapeDtypeStruct((C, total // _K, _K), jnp.float32),
        scratch_types=[
            pltpu.VMEM((N,), jnp.float32),
            pltpu.VMEM((2, R, _K), jnp.int32),
            pltpu.VMEM((2, R, _K), jnp.float32),
            pltpu.SemaphoreType.DMA,
            pltpu.SemaphoreType.DMA,
            pltpu.SemaphoreType.DMA,
        ],
        compiler_params=pltpu.CompilerParams(needs_layout_passes=False),
    )
    def body(pts_hbm, idx_hbm, out_hbm, plane_v, idx_v, vals_v,
             sem_a, sem_i, sem_o):
        wid = lax.axis_index("s") * NC + lax.axis_index("c")

        def task(c, b, slot):
            cp_a = pltpu.async_copy(pts_hbm.at[c, b], plane_v, sem_a)
            cp_i = pltpu.async_copy(
                idx_hbm.at[pl.ds(b * R, R)], idx_v.at[slot], sem_i)
            cp_i.wait()
            cp_a.wait()
            for r in range(R):
                for k in range(_K // _LANES):
                    sl = pl.ds(k * _LANES, _LANES)
                    ids = idx_v[slot, r, sl]
                    vals_v[slot, r, sl] = plsc.load_gather(plane_v, [ids])
            return pltpu.async_copy(
                vals_v.at[slot], out_hbm.at[c].at[pl.ds(b * R, R)], sem_o)

        w1 = task(wid // B, wid % B, 0)

        @pl.when(wid < C * B - NW)
        def _():
            t = wid + NW
            task(t // B, t % B, 1).wait()

        w1.wait()

    return body


def kernel(points, fps_idx):
    B, N, C = points.shape
    S = fps_idx.shape[1]

    pts3 = jnp.transpose(points, (2, 0, 1))
    idx2d = fps_idx.reshape(B * S // _K, _K).astype(jnp.int32)

    out = _sc_gather(B, N, C, S)(pts3, idx2d)
    return jnp.transpose(out.reshape(C, B, S), (1, 2, 0))


# speedup vs baseline: 151.5107x; 1.0629x over previous
import functools

import jax
import jax.numpy as jnp
from jax import lax
from jax.experimental import pallas as pl
from jax.experimental.pallas import tpu as pltpu
from jax.experimental.pallas import tpu_sc as plsc

_LANES = 16


def _sc_gather(B, N, C, S):
    mesh = plsc.VectorSubcoreMesh(core_axis_name="c", subcore_axis_name="s")
    NC = 2
    NW = 32

    @functools.partial(
        pl.kernel,
        mesh=mesh,
        out_type=jax.ShapeDtypeStruct((C, B, S), jnp.float32),
        scratch_types=[
            pltpu.VMEM((N,), jnp.float32),
            pltpu.VMEM((2, S), jnp.int32),
            pltpu.VMEM((2, S), jnp.float32),
            pltpu.SemaphoreType.DMA,
            pltpu.SemaphoreType.DMA,
            pltpu.SemaphoreType.DMA,
        ],
        compiler_params=pltpu.CompilerParams(needs_layout_passes=False),
    )
    def body(pts_hbm, idx_hbm, out_hbm, plane_v, idx_v, vals_v,
             sem_a, sem_i, sem_o):
        wid = lax.axis_index("s") * NC + lax.axis_index("c")

        def task(c, b, slot):
            cp_a = pltpu.async_copy(pts_hbm.at[c, b], plane_v, sem_a)
            cp_i = pltpu.async_copy(idx_hbm.at[b], idx_v.at[slot], sem_i)
            cp_i.wait()
            cp_a.wait()
            for g in range(S // _LANES):
                sl = pl.ds(g * _LANES, _LANES)
                ids = idx_v[slot, sl]
                vals_v[slot, sl] = plsc.load_gather(plane_v, [ids])
            return pltpu.async_copy(
                vals_v.at[slot], out_hbm.at[c, b], sem_o)

        w1 = task(wid // B, wid % B, 0)

        @pl.when(wid < C * B - NW)
        def _():
            t = wid + NW
            task(t // B, t % B, 1).wait()

        w1.wait()

    return body


def kernel(points, fps_idx):
    B, N, C = points.shape
    S = fps_idx.shape[1]

    pts3 = jnp.transpose(points, (2, 0, 1))

    out = _sc_gather(B, N, C, S)(pts3, fps_idx.astype(jnp.int32))
    return jnp.transpose(out, (1, 2, 0))
